# baseline (device time: 48616 ns/iter reference)
import jax
import jax.numpy as jnp
from jax import lax
from jax.experimental import pallas as pl
from jax.experimental.pallas import tpu as pltpu

N_DEV = 8
N_GROUPS = 12
ORDERS = tuple(((0, 1, 2), (1, 2, 0), (2, 0, 1))[g % 3] for g in range(N_GROUPS))


def kernel(A, B):
    m, k = A.shape
    k2, n = B.shape
    assert k == k2
    g_rows = m // N_GROUPS
    h_rows = g_rows // 2
    c_rows = g_rows // N_DEV

    def body(a_ref, b_ref, out_ref, mir_ref,
             rs0, rs1, rs2,
             rs_send, rs_recv, ag_send, ag_recv):
        p = lax.axis_index("i")
        plane = lax.rem(p, 4)
        zc = lax.div(p, 4)
        bx = jnp.where((plane == 1) | (plane == 2), 1, 0)
        by = jnp.where(plane >= 2, 1, 0)
        bz = zc
        nx = jnp.bitwise_xor(p, 1)
        ny = 4 * zc + (3 - plane)
        nz = jnp.bitwise_xor(p, 4)
        ax = ((nx, bx), (ny, by), (nz, bz))

        barrier_sem = pltpu.get_barrier_semaphore()
        for nbr, _ in ax:
            pl.semaphore_signal(
                barrier_sem, inc=1,
                device_id=(nbr,), device_id_type=pl.DeviceIdType.MESH,
            )

        rs_comm = (rs0, rs1, rs2)

        def matmul_rows(base, rows):
            out_ref[pl.ds(base, rows), :] = jnp.dot(
                a_ref[pl.ds(base, rows), :], b_ref[:, :],
                preferred_element_type=jnp.float32,
            )

        def start_rs(g, s, rb_g):
            blk = h_rows >> s
            nbr, bit = ax[ORDERS[g][s]]
            sb = rb_g + (1 - bit) * blk
            mir_ref[pl.ds(sb, blk), :] = out_ref[pl.ds(sb, blk), :].astype(
                jnp.bfloat16
            )
            rdma = pltpu.make_async_remote_copy(
                src_ref=mir_ref.at[pl.ds(sb, blk), :],
                dst_ref=rs_comm[s].at[g],
                send_sem=rs_send.at[s, g],
                recv_sem=rs_recv.at[s, g],
                device_id=(nbr,),
                device_id_type=pl.DeviceIdType.MESH,
            )
            rdma.start()
            return rdma, rb_g + bit * blk

        def start_ag(g, j, vb_g):
            blk = c_rows << j
            nbr, bit = ax[ORDERS[g][2 - j]]
            rdma = pltpu.make_async_remote_copy(
                src_ref=mir_ref.at[pl.ds(vb_g, blk), :],
                dst_ref=mir_ref.at[pl.ds(vb_g, blk), :],
                send_sem=ag_send.at[j, g],
                recv_sem=ag_recv.at[j, g],
                device_id=(nbr,),
                device_id_type=pl.DeviceIdType.MESH,
            )
            rdma.start()
            return rdma, bit

        rb = [g_rows * g for g in range(N_GROUPS)]
        rdmas = [None] * N_GROUPS
        for g in range(N_GROUPS):
            _, bit = ax[ORDERS[g][0]]
            matmul_rows(rb[g] + (1 - bit) * h_rows, h_rows)
            if g == 0:
                pl.semaphore_wait(barrier_sem, 3)
            rdmas[g], rb[g] = start_rs(g, 0, rb[g])
        for g in range(N_GROUPS):
            matmul_rows(rb[g], h_rows)

        for s in range(3):
            blk = h_rows >> s
            for g in range(N_GROUPS):
                rdmas[g].wait()
                out_ref[pl.ds(rb[g], blk), :] += rs_comm[s][
                    g, :, :
                ].astype(jnp.float32)
                if s < 2:
                    rdmas[g], rb[g] = start_rs(g, s + 1, rb[g])
                else:
                    mir_ref[pl.ds(rb[g], c_rows), :] = out_ref[
                        pl.ds(rb[g], c_rows), :
                    ].astype(jnp.bfloat16)
                    rdmas[g], _ = start_ag(g, 0, rb[g])

        vb = rb
        for j in range(3):
            blk = c_rows << j
            for g in range(N_GROUPS):
                rdmas[g].wait()
                _, bit = ax[ORDERS[g][2 - j]]
                pvb = vb[g] + (1 - 2 * bit) * blk
                vb[g] = vb[g] - bit * blk
                if j < 2:
                    rdmas[g], _ = start_ag(g, j + 1, vb[g])
                out_ref[pl.ds(pvb, blk), :] = mir_ref[
                    pl.ds(pvb, blk), :
                ].astype(jnp.float32)

    return pl.pallas_call(
        body,
        out_shape=jax.ShapeDtypeStruct((m, n), jnp.float32),
        in_specs=[
            pl.BlockSpec(memory_space=pltpu.VMEM),
            pl.BlockSpec(memory_space=pltpu.VMEM),
        ],
        out_specs=pl.BlockSpec(memory_space=pltpu.VMEM),
        scratch_shapes=[
            pltpu.VMEM((m, n), jnp.bfloat16),
            pltpu.VMEM((N_GROUPS, h_rows, n), jnp.bfloat16),
            pltpu.VMEM((N_GROUPS, h_rows // 2, n), jnp.bfloat16),
            pltpu.VMEM((N_GROUPS, h_rows // 4, n), jnp.bfloat16),
            pltpu.SemaphoreType.DMA((3, N_GROUPS)),
            pltpu.SemaphoreType.DMA((3, N_GROUPS)),
            pltpu.SemaphoreType.DMA((3, N_GROUPS)),
            pltpu.SemaphoreType.DMA((3, N_GROUPS)),
        ],
        compiler_params=pltpu.CompilerParams(collective_id=0),
    )(A, B)


# device time: 45126 ns/iter; 1.0773x vs baseline; 1.0773x over previous
import jax
import jax.numpy as jnp
from jax import lax
from jax.experimental import pallas as pl
from jax.experimental.pallas import tpu as pltpu

N_DEV = 8
N_GROUPS = 6
ORDERS = tuple(((0, 1, 2), (1, 2, 0), (2, 0, 1))[g % 3] for g in range(N_GROUPS))


def kernel(A, B):
    m, k = A.shape
    k2, n = B.shape
    assert k == k2
    g_rows = m // N_GROUPS
    h_rows = g_rows // 2
    q_rows = g_rows // 4

    def body(a_ref, b_ref, out_ref, mir_ref,
             rs0, rs1, rs2,
             send_sems, recv_sems):
        p = lax.axis_index("i")
        plane = lax.rem(p, 4)
        zc = lax.div(p, 4)
        bx = jnp.where((plane == 1) | (plane == 2), 1, 0)
        by = jnp.where(plane >= 2, 1, 0)
        bz = zc
        nx = jnp.bitwise_xor(p, 1)
        ny = 4 * zc + (3 - plane)
        nz = jnp.bitwise_xor(p, 4)
        ax = ((nx, bx), (ny, by), (nz, bz))

        barrier_sem = pltpu.get_barrier_semaphore()
        for nbr, _ in ax:
            pl.semaphore_signal(
                barrier_sem, inc=1,
                device_id=(nbr,), device_id_type=pl.DeviceIdType.MESH,
            )

        rs_comm = (rs0, rs1, rs2)

        def matmul_rows(base, rows):
            out_ref[pl.ds(base, rows), :] = jnp.dot(
                a_ref[pl.ds(base, rows), :], b_ref[:, :],
                preferred_element_type=jnp.float32,
            )

        def to_mirror(base, rows):
            mir_ref[pl.ds(base, rows), :] = out_ref[pl.ds(base, rows), :].astype(
                jnp.bfloat16
            )

        def exchange(g, step, axis, src_base, blk, dst_ref):
            nbr, _ = ax[axis]
            rdma = pltpu.make_async_remote_copy(
                src_ref=mir_ref.at[pl.ds(src_base, blk), :],
                dst_ref=dst_ref,
                send_sem=send_sems.at[step, g],
                recv_sem=recv_sems.at[step, g],
                device_id=(nbr,),
                device_id_type=pl.DeviceIdType.MESH,
            )
            rdma.start()
            return rdma

        rb = [g_rows * g for g in range(N_GROUPS)]
        rdmas = [None] * N_GROUPS
        for g in range(N_GROUPS):
            a1 = ORDERS[g][0]
            _, bit = ax[a1]
            sb = rb[g] + (1 - bit) * h_rows
            matmul_rows(sb, h_rows)
            to_mirror(sb, h_rows)
            if g == 0:
                pl.semaphore_wait(barrier_sem, 3)
            rdmas[g] = exchange(g, 0, a1, sb, h_rows, rs0.at[g])
            rb[g] = rb[g] + bit * h_rows
        for g in range(N_GROUPS):
            matmul_rows(rb[g], h_rows)

        for g in range(N_GROUPS):
            a2 = ORDERS[g][1]
            _, bit = ax[a2]
            rdmas[g].wait()
            out_ref[pl.ds(rb[g], h_rows), :] += rs0[g, :, :].astype(jnp.float32)
            sb = rb[g] + (1 - bit) * q_rows
            to_mirror(sb, q_rows)
            rdmas[g] = exchange(g, 1, a2, sb, q_rows, rs1.at[g])
            rb[g] = rb[g] + bit * q_rows

        for g in range(N_GROUPS):
            a3 = ORDERS[g][2]
            rdmas[g].wait()
            out_ref[pl.ds(rb[g], q_rows), :] += rs1[g, :, :].astype(jnp.float32)
            to_mirror(rb[g], q_rows)
            rdmas[g] = exchange(g, 2, a3, rb[g], q_rows, rs2.at[g])

        pvb = [None] * N_GROUPS
        for g in range(N_GROUPS):
            a2 = ORDERS[g][1]
            _, bit = ax[a2]
            rdmas[g].wait()
            out_ref[pl.ds(rb[g], q_rows), :] += rs2[g, :, :].astype(jnp.float32)
            to_mirror(rb[g], q_rows)
            rdmas[g] = exchange(
                g, 3, a2, rb[g], q_rows,
                mir_ref.at[pl.ds(rb[g], q_rows), :],
            )
            pvb[g] = rb[g] + (1 - 2 * bit) * q_rows
            rb[g] = rb[g] - bit * q_rows

        pvb2 = [None] * N_GROUPS
        for g in range(N_GROUPS):
            a1 = ORDERS[g][0]
            _, bit = ax[a1]
            rdmas[g].wait()
            rdmas[g] = exchange(
                g, 4, a1, rb[g], h_rows,
                mir_ref.at[pl.ds(rb[g], h_rows), :],
            )
            pvb2[g] = rb[g] + (1 - 2 * bit) * h_rows
            out_ref[pl.ds(pvb[g], q_rows), :] = mir_ref[
                pl.ds(pvb[g], q_rows), :
            ].astype(jnp.float32)

        for g in range(N_GROUPS):
            rdmas[g].wait()
            out_ref[pl.ds(pvb2[g], h_rows), :] = mir_ref[
                pl.ds(pvb2[g], h_rows), :
            ].astype(jnp.float32)

    return pl.pallas_call(
        body,
        out_shape=jax.ShapeDtypeStruct((m, n), jnp.float32),
        in_specs=[
            pl.BlockSpec(memory_space=pltpu.VMEM),
            pl.BlockSpec(memory_space=pltpu.VMEM),
        ],
        out_specs=pl.BlockSpec(memory_space=pltpu.VMEM),
        scratch_shapes=[
            pltpu.VMEM((m, n), jnp.bfloat16),
            pltpu.VMEM((N_GROUPS, h_rows, n), jnp.bfloat16),
            pltpu.VMEM((N_GROUPS, q_rows, n), jnp.bfloat16),
            pltpu.VMEM((N_GROUPS, q_rows, n), jnp.bfloat16),
            pltpu.SemaphoreType.DMA((5, N_GROUPS)),
            pltpu.SemaphoreType.DMA((5, N_GROUPS)),
        ],
        compiler_params=pltpu.CompilerParams(collective_id=0),
    )(A, B)


# device time: 45094 ns/iter; 1.0781x vs baseline; 1.0007x over previous
import jax
import jax.numpy as jnp
from jax import lax
from jax.experimental import pallas as pl
from jax.experimental.pallas import tpu as pltpu

N_DEV = 8
N_GROUPS = 6
ORDERS = tuple(((0, 1, 2), (1, 2, 0), (2, 0, 1))[g % 3] for g in range(N_GROUPS))


def kernel(A, B):
    m, k = A.shape
    k2, n = B.shape
    assert k == k2
    g_rows = m // N_GROUPS
    h_rows = g_rows // 2
    q_rows = g_rows // 4

    def body(a_ref, b_ref, out_ref, mir_ref,
             rs0, rs1, rs2,
             send_sems, recv_sems):
        p = lax.axis_index("i")
        plane = lax.rem(p, 4)
        zc = lax.div(p, 4)
        bx = jnp.where((plane == 1) | (plane == 2), 1, 0)
        by = jnp.where(plane >= 2, 1, 0)
        bz = zc
        nx = jnp.bitwise_xor(p, 1)
        ny = 4 * zc + (3 - plane)
        nz = jnp.bitwise_xor(p, 4)
        ax = ((nx, bx), (ny, by), (nz, bz))

        barrier_sem = pltpu.get_barrier_semaphore()
        for nbr, _ in ax:
            pl.semaphore_signal(
                barrier_sem, inc=1,
                device_id=(nbr,), device_id_type=pl.DeviceIdType.MESH,
            )

        rs_comm = (rs0, rs1, rs2)

        def matmul_rows(base, rows):
            out_ref[pl.ds(base, rows), :] = jnp.dot(
                a_ref[pl.ds(base, rows), :], b_ref[:, :],
                preferred_element_type=jnp.float32,
            )

        def to_mirror(base, rows):
            mir_ref[pl.ds(base, rows), :] = out_ref[pl.ds(base, rows), :].astype(
                jnp.bfloat16
            )

        def exchange(g, step, axis, src_base, blk, dst_ref):
            nbr, _ = ax[axis]
            rdma = pltpu.make_async_remote_copy(
                src_ref=mir_ref.at[pl.ds(src_base, blk), :],
                dst_ref=dst_ref,
                send_sem=send_sems.at[step, g],
                recv_sem=recv_sems.at[step, g],
                device_id=(nbr,),
                device_id_type=pl.DeviceIdType.MESH,
            )
            rdma.start()
            return rdma

        rb = [g_rows * g for g in range(N_GROUPS)]
        rdmas = [None] * N_GROUPS
        for g in range(N_GROUPS):
            a1 = ORDERS[g][0]
            _, bit = ax[a1]
            sb = rb[g] + (1 - bit) * h_rows
            mir_ref[pl.ds(sb, h_rows), :] = jnp.dot(
                a_ref[pl.ds(sb, h_rows), :], b_ref[:, :],
                preferred_element_type=jnp.float32,
            ).astype(jnp.bfloat16)
            if g == 0:
                pl.semaphore_wait(barrier_sem, 3)
            rdmas[g] = exchange(g, 0, a1, sb, h_rows, rs0.at[g])
            rb[g] = rb[g] + bit * h_rows
        for g in range(N_GROUPS):
            matmul_rows(rb[g], h_rows)

        for g in range(N_GROUPS):
            a2 = ORDERS[g][1]
            _, bit = ax[a2]
            rdmas[g].wait()
            soff = (1 - bit) * q_rows
            sb = rb[g] + soff
            kb = rb[g] + (q_rows - soff)
            mir_ref[pl.ds(sb, q_rows), :] = (
                out_ref[pl.ds(sb, q_rows), :]
                + rs0[g, pl.ds(soff, q_rows), :].astype(jnp.float32)
            ).astype(jnp.bfloat16)
            rdmas[g] = exchange(g, 1, a2, sb, q_rows, rs1.at[g])
            out_ref[pl.ds(kb, q_rows), :] += rs0[
                g, pl.ds(q_rows - soff, q_rows), :
            ].astype(jnp.float32)
            rb[g] = rb[g] + bit * q_rows

        for g in range(N_GROUPS):
            a3 = ORDERS[g][2]
            rdmas[g].wait()
            s = out_ref[pl.ds(rb[g], q_rows), :] + rs1[g, :, :].astype(
                jnp.float32
            )
            out_ref[pl.ds(rb[g], q_rows), :] = s
            mir_ref[pl.ds(rb[g], q_rows), :] = s.astype(jnp.bfloat16)
            rdmas[g] = exchange(g, 2, a3, rb[g], q_rows, rs2.at[g])

        pvb = [None] * N_GROUPS
        for g in range(N_GROUPS):
            a2 = ORDERS[g][1]
            _, bit = ax[a2]
            rdmas[g].wait()
            s = out_ref[pl.ds(rb[g], q_rows), :] + rs2[g, :, :].astype(
                jnp.float32
            )
            out_ref[pl.ds(rb[g], q_rows), :] = s
            mir_ref[pl.ds(rb[g], q_rows), :] = s.astype(jnp.bfloat16)
            rdmas[g] = exchange(
                g, 3, a2, rb[g], q_rows,
                mir_ref.at[pl.ds(rb[g], q_rows), :],
            )
            pvb[g] = rb[g] + (1 - 2 * bit) * q_rows
            rb[g] = rb[g] - bit * q_rows

        pvb2 = [None] * N_GROUPS
        for g in range(N_GROUPS):
            a1 = ORDERS[g][0]
            _, bit = ax[a1]
            rdmas[g].wait()
            rdmas[g] = exchange(
                g, 4, a1, rb[g], h_rows,
                mir_ref.at[pl.ds(rb[g], h_rows), :],
            )
            pvb2[g] = rb[g] + (1 - 2 * bit) * h_rows
            out_ref[pl.ds(pvb[g], q_rows), :] = mir_ref[
                pl.ds(pvb[g], q_rows), :
            ].astype(jnp.float32)

        for g in range(N_GROUPS):
            rdmas[g].wait()
            out_ref[pl.ds(pvb2[g], h_rows), :] = mir_ref[
                pl.ds(pvb2[g], h_rows), :
            ].astype(jnp.float32)

    return pl.pallas_call(
        body,
        out_shape=jax.ShapeDtypeStruct((m, n), jnp.float32),
        in_specs=[
            pl.BlockSpec(memory_space=pltpu.VMEM),
            pl.BlockSpec(memory_space=pltpu.VMEM),
        ],
        out_specs=pl.BlockSpec(memory_space=pltpu.VMEM),
        scratch_shapes=[
            pltpu.VMEM((m, n), jnp.bfloat16),
            pltpu.VMEM((N_GROUPS, h_rows, n), jnp.bfloat16),
            pltpu.VMEM((N_GROUPS, q_rows, n), jnp.bfloat16),
            pltpu.VMEM((N_GROUPS, q_rows, n), jnp.bfloat16),
            pltpu.SemaphoreType.DMA((5, N_GROUPS)),
            pltpu.SemaphoreType.DMA((5, N_GROUPS)),
        ],
        compiler_params=pltpu.CompilerParams(collective_id=0),
    )(A, B)
